# Initial kernel scaffold; baseline (speedup 1.0000x reference)
import jax
import jax.numpy as jnp
from jax.experimental import pallas as pl


def kernel(x, W_wave, b_wave, wave_gamma, W_kernel, b_kernel, kernel_gamma, W_exp, b_exp, exp_gamma, W_out):
    raise NotImplementedError("write your pallas kernel here")


# speedup vs baseline: 15.5915x; 15.5915x over previous
import functools

import jax
import jax.numpy as jnp
from jax.experimental import pallas as pl

B, L, C = 2, 2048, 768
H, K, HALF_S = 12, 32, 4
S = 2 * HALF_S + 1
D = C // H
MAX_FREQ, MIN_FREQ = 16.0, 1.0
MAX_RECEPTIVE = HALF_S * MAX_FREQ

T = 256
HALO = 128
W = T + 2 * HALO


def _silu(v):
    return v * jax.nn.sigmoid(v)


def _body(x_ref, Ww_ref, bw_ref, gw_ref, Wk_ref, bk_ref, gk_ref,
          We_ref, be_ref, ge_ref, Wo_ref, out_ref):
    blk = pl.program_id(1)
    l0 = blk * T
    xb = x_ref[0, pl.ds(l0 + HALO, T), :]

    pw = jnp.dot(xb, Ww_ref[...], preferred_element_type=jnp.float32) + bw_ref[...][None, :]
    var_w = jnp.sum(pw * pw, axis=-1, keepdims=True) / (2 * H)
    wave = _silu(gw_ref[...][None, :] * (pw * jax.lax.rsqrt(var_w + 1e-6)))
    freq = jax.nn.sigmoid(wave[:, :H]) * (MAX_FREQ - MIN_FREQ) + MIN_FREQ
    phase = jnp.tanh(wave[:, H:2 * H]) * MAX_FREQ

    pk = jnp.dot(xb, Wk_ref[...], preferred_element_type=jnp.float32) + bk_ref[...][None, :]
    var_k = jnp.sum(pk * pk, axis=-1, keepdims=True) / (H * K)
    km = _silu(gk_ref[...][None, :] * (pk * jax.lax.rsqrt(var_k + 1e-6)))

    pe = jnp.dot(xb, We_ref[...], preferred_element_type=jnp.float32) + be_ref[...][None, :]
    ve = pe[:, 0:1]
    ve_n = ge_ref[...][0:1][None, :] * (ve * jax.lax.rsqrt(ve * ve + 1e-6))
    exponent = jax.nn.sigmoid(ve_n) * 3.5 + 0.5

    centers = (l0 + jax.lax.broadcasted_iota(jnp.int32, (T, 1), 0)).astype(jnp.float32)
    col = jax.lax.broadcasted_iota(jnp.int32, (T, W), 1)
    iota_k = jax.lax.broadcasted_iota(jnp.int32, (T, K), 1)

    outs = []
    for h in range(H):
        fh = freq[:, h:h + 1]
        ph = phase[:, h:h + 1]
        kmh = km[:, h * K:(h + 1) * K]
        M = jnp.zeros((T, W), jnp.float32)
        for s in range(S):
            stride = float(s - HALF_S)
            rel = stride * fh
            sp = centers + rel + ph
            valid = ((sp >= 0) & (sp < L)).astype(jnp.float32)
            pos_c = jnp.clip(sp, 0.0, L - 1.001)
            sfloor = jnp.clip(jnp.floor(pos_c).astype(jnp.int32), 0, L - 1)
            frac = pos_c - sfloor.astype(jnp.float32)
            nd = jnp.abs(rel) / L
            pwr = jnp.exp(-exponent * jnp.log1p(nd))
            np_ = jnp.clip(jnp.abs(rel) / MAX_RECEPTIVE, 0.0, 1.0)
            idx_f = np_ * (K - 1)
            idxf = jnp.clip(idx_f.astype(jnp.int32), 0, K - 2)
            w_ce = idx_f - idxf.astype(jnp.float32)
            kf = jnp.sum(jnp.where(iota_k == idxf, kmh, 0.0), axis=-1, keepdims=True)
            kc = jnp.sum(jnp.where(iota_k == idxf + 1, kmh, 0.0), axis=-1, keepdims=True)
            ker = (kf * (1.0 - w_ce) + kc * w_ce) * pwr * valid
            wf = ker * (1.0 - frac)
            wc = ker * frac
            g = col - (sfloor + (HALO - l0))
            M = M + jnp.where(g == 0, wf, jnp.where(g == 1, wc, 0.0))
        xh = x_ref[0, pl.ds(l0, W), h * D:(h + 1) * D]
        outs.append(jnp.dot(M, xh, preferred_element_type=jnp.float32))
    acc = jnp.concatenate(outs, axis=-1)
    out_ref[0] = jnp.dot(acc, Wo_ref[...], preferred_element_type=jnp.float32)


@jax.jit
def kernel(x, W_wave, b_wave, wave_gamma, W_kernel, b_kernel, kernel_gamma,
           W_exp, b_exp, exp_gamma, W_out):
    f32 = jnp.float32
    x_pad = jnp.pad(x, ((0, 0), (HALO, HALO), (0, 0)))
    Ww = jnp.zeros((C, 128), f32).at[:, :2 * H].set(W_wave.T)
    bw = jnp.zeros((128,), f32).at[:2 * H].set(b_wave)
    gw = jnp.zeros((128,), f32).at[:2 * H].set(wave_gamma)
    We = jnp.zeros((C, 128), f32).at[:, :1].set(W_exp.T)
    be = jnp.zeros((128,), f32).at[:1].set(b_exp)
    ge = jnp.zeros((128,), f32).at[:1].set(exp_gamma)

    grid = (B, L // T)
    full = lambda shape: pl.BlockSpec(shape, lambda b, i: (0,) * len(shape))
    out = pl.pallas_call(
        _body,
        grid=grid,
        in_specs=[
            pl.BlockSpec((1, L + 2 * HALO, C), lambda b, i: (b, 0, 0)),
            full((C, 128)), full((128,)), full((128,)),
            full((C, H * K)), full((H * K,)), full((H * K,)),
            full((C, 128)), full((128,)), full((128,)),
            full((C, C)),
        ],
        out_specs=pl.BlockSpec((1, T, C), lambda b, i: (b, i, 0)),
        out_shape=jax.ShapeDtypeStruct((B, L, C), f32),
    )(x_pad, Ww, bw, gw, W_kernel.T, b_kernel, kernel_gamma, We, be, ge, W_out.T)
    return out
